# baseline (speedup 1.0000x reference)
import functools

import jax
import jax.numpy as jnp
from jax import lax
from jax.experimental import pallas as pl
from jax.experimental.pallas import tpu as pltpu
from jax.experimental.pallas import tpu_sc as plsc

_K = 8
_E = 64
_T = 1024
_NW = 32

_SORT8 = [(0, 1), (2, 3), (4, 5), (6, 7),
          (0, 2), (1, 3), (4, 6), (5, 7),
          (1, 2), (5, 6),
          (0, 4), (1, 5), (2, 6), (3, 7),
          (2, 4), (3, 5),
          (1, 2), (3, 4), (5, 6)]
_BITONIC8 = [(0, 4), (1, 5), (2, 6), (3, 7),
             (0, 2), (1, 3), (4, 6), (5, 7),
             (0, 1), (2, 3), (4, 5), (6, 7)]


def _probs_block(x_ref, w_ref, probs_ref):
    x = x_ref[...]
    w = w_ref[...]
    logits = lax.dot_general(
        w, x, (((1,), (1,)), ((), ())), preferred_element_type=jnp.float32
    )
    m = jnp.max(logits, axis=0, keepdims=True)
    e = jnp.exp(logits - m)
    s = jnp.sum(e, axis=0, keepdims=True)
    probs_ref[...] = (e * (1.0 / s))[None]


def _ce(lst, i, j):
    av, ai = lst[i]
    bv, bi = lst[j]
    m = av >= bv
    lst[i] = (jnp.where(m, av, bv), jnp.where(m, ai, bi))
    lst[j] = (jnp.where(m, bv, av), jnp.where(m, bi, ai))


def _sc_top8(probs_hbm, scores_hbm, idx_hbm, buf, sco, sio):
    wid = lax.axis_index("s") * 2 + lax.axis_index("c")
    pltpu.sync_copy(probs_hbm.at[wid], buf)

    def group(g, carry):
        base = g * 16
        vs = [buf[j, pl.ds(base, 16)] for j in range(_E)]
        tops = []
        for grp in range(8):
            lst = [(vs[8 * grp + j], jnp.full((16,), 8 * grp + j, jnp.int32))
                   for j in range(8)]
            for (i, j) in _SORT8:
                _ce(lst, i, j)
            tops.append(lst)
        while len(tops) > 1:
            nxt = []
            for t in range(0, len(tops), 2):
                a, b = tops[t], tops[t + 1]
                c = []
                for i in range(8):
                    av, ai = a[i]
                    bv, bi = b[7 - i]
                    m = av >= bv
                    c.append((jnp.where(m, av, bv), jnp.where(m, ai, bi)))
                for (i, j) in _BITONIC8:
                    _ce(c, i, j)
                nxt.append(c)
            tops = nxt
        top = tops[0]
        for k in range(_K):
            sco[k, pl.ds(base, 16)] = top[k][0]
            sio[k, pl.ds(base, 16)] = top[k][1]
        return carry

    lax.fori_loop(0, _T // 16, group, 0)
    pltpu.sync_copy(sco, scores_hbm.at[wid])
    pltpu.sync_copy(sio, idx_hbm.at[wid])


@jax.jit
def kernel(x, W):
    n_tokens, emb = x.shape
    nb = n_tokens // _T
    probs_t = pl.pallas_call(
        _probs_block,
        grid=(nb,),
        in_specs=[
            pl.BlockSpec((_T, emb), lambda i: (i, 0)),
            pl.BlockSpec((_E, emb), lambda i: (0, 0)),
        ],
        out_specs=pl.BlockSpec((1, _E, _T), lambda i: (i, 0, 0)),
        out_shape=jax.ShapeDtypeStruct((nb, _E, _T), jnp.float32),
    )(x, W)

    sc_call = functools.partial(
        pl.kernel,
        out_type=[
            jax.ShapeDtypeStruct((nb, _K, _T), jnp.float32),
            jax.ShapeDtypeStruct((nb, _K, _T), jnp.int32),
        ],
        mesh=plsc.VectorSubcoreMesh(core_axis_name="c", subcore_axis_name="s"),
        scratch_types=[
            pltpu.VMEM((_E, _T), jnp.float32),
            pltpu.VMEM((_K, _T), jnp.float32),
            pltpu.VMEM((_K, _T), jnp.int32),
        ],
    )(_sc_top8)
    sco, sio = sc_call(probs_t)

    scores = sco.transpose(0, 2, 1).reshape(n_tokens, _K)
    idx = sio.transpose(0, 2, 1).reshape(n_tokens, _K)
    return scores, idx
